# baseline (speedup 1.0000x reference)
import functools

import jax
import jax.numpy as jnp
from jax import lax
from jax.experimental import pallas as pl
from jax.experimental.pallas import tpu as pltpu
from jax.experimental.pallas import tpu_sc as plsc


def _sc_gather_wide(table_pad, idx):
    V, DW = table_pad.shape
    B = idx.shape[0]
    NW = 32
    b_per_w = B // NW
    mesh = plsc.VectorSubcoreMesh(core_axis_name="c", subcore_axis_name="s")

    @functools.partial(
        pl.kernel,
        mesh=mesh,
        compiler_params=pltpu.CompilerParams(use_tc_tiling_on_sc=False),
        out_type=jax.ShapeDtypeStruct((B, DW), jnp.float32),
        scratch_types=[
            pltpu.VMEM((b_per_w // 2,), jnp.int32),
            pltpu.VMEM((b_per_w // 2,), jnp.int32),
            pltpu.VMEM((b_per_w, DW), jnp.float32),
            pltpu.SemaphoreType.DMA,
            pltpu.SemaphoreType.DMA,
        ],
    )
    def gather_kernel(table_hbm, idx_hbm, out_hbm, idx_a, idx_b, rows_v,
                      sem_a, sem_b):
        wid = lax.axis_index("s") * 2 + lax.axis_index("c")
        base = wid * b_per_w
        half = b_per_w // 2
        pltpu.sync_copy(idx_hbm.at[pl.ds(base, half)], idx_a)
        pltpu.sync_copy(idx_hbm.at[pl.ds(base + half, half)], idx_b)
        ca = pltpu.make_async_copy(
            table_hbm.at[idx_a], rows_v.at[pl.ds(0, half)], sem_a)
        cb = pltpu.make_async_copy(
            table_hbm.at[idx_b], rows_v.at[pl.ds(half, half)], sem_b)
        ca.start()
        cb.start()
        ca.wait()
        cb.wait()
        pltpu.sync_copy(rows_v, out_hbm.at[pl.ds(base, b_per_w)])

    return gather_kernel(table_pad, idx)


def _proj_body(emb_ref, w_ref, b_ref, out_ref):
    out_ref[...] = (
        jnp.dot(emb_ref[:, :64], w_ref[...], preferred_element_type=jnp.float32)
        + b_ref[...]
    )


def _tc_project(emb_wide, W, b):
    B = emb_wide.shape[0]
    D, N = W.shape
    BM = 4096
    return pl.pallas_call(
        _proj_body,
        grid=(B // BM,),
        in_specs=[
            pl.BlockSpec((BM, 2 * D), lambda i: (i, 0)),
            pl.BlockSpec((D, N), lambda i: (0, 0)),
            pl.BlockSpec((1, N), lambda i: (0, 0)),
        ],
        out_specs=pl.BlockSpec((BM, N), lambda i: (i, 0)),
        out_shape=jax.ShapeDtypeStruct((B, N), jnp.float32),
    )(emb_wide, W, b.reshape(1, N))


def kernel(user_embeds, user_table, W, b):
    V, D = user_table.shape
    pad_id = jnp.eye(D, 2 * D, dtype=user_table.dtype)
    table_pad = user_table @ pad_id
    emb_wide = _sc_gather_wide(table_pad, user_embeds)
    return _tc_project(emb_wide, W, b)


# speedup vs baseline: 1.0064x; 1.0064x over previous
import functools

import jax
import jax.numpy as jnp
from jax import lax
from jax.experimental import pallas as pl
from jax.experimental.pallas import tpu as pltpu
from jax.experimental.pallas import tpu_sc as plsc


def _sc_gather_wide(table_pad, idx):
    V, DW = table_pad.shape
    B = idx.shape[0]
    NW = 32
    b_per_w = B // NW
    mesh = plsc.VectorSubcoreMesh(core_axis_name="c", subcore_axis_name="s")

    @functools.partial(
        pl.kernel,
        mesh=mesh,
        compiler_params=pltpu.CompilerParams(use_tc_tiling_on_sc=False),
        out_type=jax.ShapeDtypeStruct((B, DW), jnp.float32),
        scratch_types=[
            pltpu.VMEM((b_per_w,), jnp.int32),
            pltpu.VMEM((b_per_w, DW), jnp.float32),
            pltpu.SemaphoreType.DMA,
        ],
    )
    def gather_kernel(table_hbm, idx_hbm, out_hbm, idx_v, rows_v, sem):
        wid = lax.axis_index("s") * 2 + lax.axis_index("c")
        base = wid * b_per_w
        pltpu.sync_copy(idx_hbm.at[pl.ds(base, b_per_w)], idx_v)
        pltpu.async_copy(table_hbm.at[idx_v], rows_v, sem).wait()
        pltpu.sync_copy(rows_v, out_hbm.at[pl.ds(base, b_per_w)])

    return gather_kernel(table_pad, idx)


def _proj_body(emb_ref, w_ref, b_ref, out_ref):
    out_ref[...] = (
        jnp.dot(emb_ref[:, :64], w_ref[...], preferred_element_type=jnp.float32)
        + b_ref[...]
    )


def _tc_project(emb_wide, W, b):
    B = emb_wide.shape[0]
    D, N = W.shape
    BM = 4096
    return pl.pallas_call(
        _proj_body,
        grid=(B // BM,),
        in_specs=[
            pl.BlockSpec((BM, 2 * D), lambda i: (i, 0)),
            pl.BlockSpec((D, N), lambda i: (0, 0)),
            pl.BlockSpec((1, N), lambda i: (0, 0)),
        ],
        out_specs=pl.BlockSpec((BM, N), lambda i: (i, 0)),
        out_shape=jax.ShapeDtypeStruct((B, N), jnp.float32),
    )(emb_wide, W, b.reshape(1, N))


def kernel(user_embeds, user_table, W, b):
    V, D = user_table.shape
    pad_id = jnp.eye(D, 2 * D, dtype=user_table.dtype)
    table_pad = user_table @ pad_id
    emb_wide = _sc_gather_wide(table_pad, user_embeds)
    return _tc_project(emb_wide, W, b)
